# baseline (speedup 1.0000x reference)
import functools

import jax
import jax.numpy as jnp
from jax import lax
from jax.experimental import pallas as pl
from jax.experimental.pallas import tpu as pltpu
from jax.experimental.pallas import tpu_sc as plsc

B, T = 1024, 50
NUM = 16
N_CAT = 26
VOCAB = 100000
EDIM = 32
OUT_F = NUM + N_CAT * EDIM
IP = 64
NCV = 28
NSLOT = NCV // 4
IB = NCV * IP
SROWS = 128
NSTR = IB // SROWS
XD = IB * EDIM // 128

_GI = 32


def _idx_body(cat_ref, out_ref):
    y = cat_ref[...] + lax.broadcasted_iota(jnp.int32, (_GI, N_CAT, T), 1) * VOCAB
    padt = lax.broadcasted_iota(jnp.int32, (_GI, IP - T), 1) + T
    dummy = lax.broadcasted_iota(jnp.int32, (_GI, IP), 1)
    pieces = []
    for s in range(NSLOT):
        for h in range(4):
            c = h + 4 * s
            if c < N_CAT:
                pieces.append(y[:, c, :])
                pieces.append(padt + c * VOCAB)
            else:
                pieces.append(dummy + (c - N_CAT) * IP)
    nat = jnp.concatenate(pieces, axis=1).astype(jnp.float32)
    a = lax.broadcasted_iota(jnp.int32, (4 * IP, 4 * IP), 0)
    w = lax.broadcasted_iota(jnp.int32, (4 * IP, 4 * IP), 1)
    perm = jnp.where(w == 4 * (a % IP) + a // IP, 1.0, 0.0)
    q = lax.dot_general(
        nat.reshape(_GI * NSLOT, 4 * IP), perm, (((1,), (0,)), ((), ())),
        precision=lax.Precision.HIGHEST,
        preferred_element_type=jnp.float32,
    )
    out_ref[...] = q.astype(jnp.int32).reshape(_GI * NSTR, SROWS)


def _make_sc_gather(num_workers: int):
    b_per_w = B // num_workers
    mesh = plsc.VectorSubcoreMesh(
        core_axis_name="c", subcore_axis_name="s", num_cores=2)

    @functools.partial(
        pl.kernel,
        mesh=mesh,
        compiler_params=pltpu.CompilerParams(use_tc_tiling_on_sc=False),
        out_type=jax.ShapeDtypeStruct((B, IB, EDIM), jnp.float32),
        scratch_types=[
            pltpu.VMEM((NSTR, SROWS), jnp.int32),
            pltpu.VMEM((2, IB, EDIM), jnp.float32),
            pltpu.SemaphoreType.DMA,
            pltpu.SemaphoreType.DMA,
        ],
    )
    def k(cat_hbm, tab_hbm, x_hbm, idx_v, vbuf, gsem, wsem):
        nc = plsc.get_sparse_core_info().num_cores
        wid = lax.axis_index("s") * nc + lax.axis_index("c")
        b0 = wid * b_per_w

        def load_and_fire(b, p):
            pltpu.sync_copy(cat_hbm.at[pl.ds(b * NSTR, NSTR)], idx_v)

            def fire_s(s, cc):
                pltpu.async_copy(
                    tab_hbm.at[idx_v.at[s]],
                    vbuf.at[p, pl.ds(s * SROWS, SROWS)],
                    gsem,
                )
                return cc

            lax.fori_loop(0, NSTR, fire_s, 0)

        load_and_fire(b0, 0)

        def body_b(bi, carry):
            b = b0 + bi
            p = lax.rem(bi, 2)
            q = 1 - p
            pltpu.make_async_copy(x_hbm.at[b], vbuf.at[p], gsem).wait()
            pltpu.async_copy(vbuf.at[p], x_hbm.at[b], wsem)

            @pl.when(bi < b_per_w - 1)
            def _():
                @pl.when(bi > 0)
                def _():
                    pltpu.make_async_copy(vbuf.at[q], x_hbm.at[b], wsem).wait()

                load_and_fire(b + 1, q)

            return carry

        lax.fori_loop(0, b_per_w, body_b, 0)
        pltpu.make_async_copy(vbuf.at[0], x_hbm.at[b0], wsem).wait()
        pltpu.make_async_copy(vbuf.at[1], x_hbm.at[b0], wsem).wait()

    return k


_GB = 8


def _tc_body(x_ref, num_ref, out_ref):
    v = x_ref[...]
    vt = jnp.swapaxes(v, 1, 2)
    for g in range(_GB):
        out_ref[g, 0:NUM, :] = num_ref[g]
        for c in range(N_CAT):
            h, s = c % 4, c // 4
            out_ref[g, NUM + EDIM * c:NUM + EDIM * (c + 1), :] = (
                vt[g, EDIM * h:EDIM * (h + 1), IP * s:IP * s + T])


def kernel(numerical, categorical, tables):
    info = plsc.get_sparse_core_info()
    nw = info.num_cores * info.num_subcores
    tab_flat = tables.reshape(N_CAT * VOCAB, EDIM)
    cat_prep = pl.pallas_call(
        _idx_body,
        grid=(B // _GI,),
        in_specs=[pl.BlockSpec((_GI, N_CAT, T), lambda i: (i, 0, 0))],
        out_specs=pl.BlockSpec((_GI * NSTR, SROWS), lambda i: (i, 0)),
        out_shape=jax.ShapeDtypeStruct((B * NSTR, SROWS), jnp.int32),
    )(categorical)
    x = _make_sc_gather(nw)(cat_prep, tab_flat)
    out = pl.pallas_call(
        _tc_body,
        grid=(B // _GB,),
        in_specs=[
            pl.BlockSpec((_GB, XD, 128), lambda i: (i, 0, 0)),
            pl.BlockSpec((_GB, NUM, T), lambda i: (i, 0, 0)),
        ],
        out_specs=pl.BlockSpec((_GB, OUT_F, T), lambda i: (i, 0, 0)),
        out_shape=jax.ShapeDtypeStruct((B, OUT_F, T), jnp.float32),
    )(x.reshape(B, XD, 128), numerical)
    return out


# speedup vs baseline: 1.5166x; 1.5166x over previous
import functools

import jax
import jax.numpy as jnp
from jax import lax
from jax.experimental import pallas as pl
from jax.experimental.pallas import tpu as pltpu
from jax.experimental.pallas import tpu_sc as plsc

B, T = 1024, 50
NUM = 16
N_CAT = 26
VOCAB = 100000
EDIM = 32
OUT_F = NUM + N_CAT * EDIM
IP = 64
NCV = 28
NSLOT = NCV // 4
IB = NCV * IP
SROWS = 128
NSTR = IB // SROWS
XD = IB * EDIM // 128

_GI = 32


def _idx_body(cat_ref, out_ref):
    y = cat_ref[...] + lax.broadcasted_iota(jnp.int32, (_GI, N_CAT, T), 1) * VOCAB
    padt = lax.broadcasted_iota(jnp.int32, (_GI, IP - T), 1) + T
    dummy = lax.broadcasted_iota(jnp.int32, (_GI, IP), 1)
    pieces = []
    for s in range(NSLOT):
        for h in range(4):
            c = h + 4 * s
            if c < N_CAT:
                pieces.append(y[:, c, :])
                pieces.append(padt + c * VOCAB)
            else:
                pieces.append(dummy + (c - N_CAT) * IP)
    nat = jnp.concatenate(pieces, axis=1).astype(jnp.float32)
    a = lax.broadcasted_iota(jnp.int32, (4 * IP, 4 * IP), 0)
    w = lax.broadcasted_iota(jnp.int32, (4 * IP, 4 * IP), 1)
    perm = jnp.where(w == 4 * (a % IP) + a // IP, 1.0, 0.0)
    q = lax.dot_general(
        nat.reshape(_GI * NSLOT, 4 * IP), perm, (((1,), (0,)), ((), ())),
        precision=lax.Precision.HIGHEST,
        preferred_element_type=jnp.float32,
    )
    out_ref[...] = q.astype(jnp.int32).reshape(_GI * NSTR, SROWS)


def _make_sc_gather(num_workers: int):
    b_per_w = B // num_workers
    mesh = plsc.VectorSubcoreMesh(
        core_axis_name="c", subcore_axis_name="s", num_cores=2)

    @functools.partial(
        pl.kernel,
        mesh=mesh,
        compiler_params=pltpu.CompilerParams(use_tc_tiling_on_sc=False),
        out_type=jax.ShapeDtypeStruct((B * IB, EDIM), jnp.float32),
        scratch_types=[
            pltpu.VMEM((NSTR, SROWS), jnp.int32),
            pltpu.VMEM((2, IB, EDIM), jnp.float32),
            pltpu.SemaphoreType.DMA,
            pltpu.SemaphoreType.DMA,
        ],
    )
    def k(cat_hbm, tab_hbm, x_hbm, idx_v, vbuf, gsem, wsem):
        nc = plsc.get_sparse_core_info().num_cores
        wid = lax.axis_index("s") * nc + lax.axis_index("c")
        b0 = wid * b_per_w

        def load_and_fire(b, p):
            pltpu.sync_copy(cat_hbm.at[pl.ds(b * NSTR, NSTR)], idx_v)

            def fire_s(s, cc):
                pltpu.async_copy(
                    tab_hbm.at[idx_v.at[s]],
                    vbuf.at[p, pl.ds(s * SROWS, SROWS)],
                    gsem,
                )
                return cc

            lax.fori_loop(0, NSTR, fire_s, 0)

        load_and_fire(b0, 0)

        def body_b(bi, carry):
            b = b0 + bi
            p = lax.rem(bi, 2)
            q = 1 - p
            xslc = x_hbm.at[pl.ds(b * IB, IB)]
            pltpu.make_async_copy(xslc, vbuf.at[p], gsem).wait()
            pltpu.async_copy(vbuf.at[p], xslc, wsem)

            @pl.when(bi < b_per_w - 1)
            def _():
                @pl.when(bi > 0)
                def _():
                    pltpu.make_async_copy(vbuf.at[q], xslc, wsem).wait()

                load_and_fire(b + 1, q)

            return carry

        lax.fori_loop(0, b_per_w, body_b, 0)
        xsl0 = x_hbm.at[pl.ds(b0 * IB, IB)]
        pltpu.make_async_copy(vbuf.at[0], xsl0, wsem).wait()
        pltpu.make_async_copy(vbuf.at[1], xsl0, wsem).wait()

    return k


_GB = 8


def _tc_body(x_ref, num_ref, out_ref):
    v = x_ref[...].reshape(_GB, XD, 128)
    vt = jnp.swapaxes(v, 1, 2)
    for g in range(_GB):
        out_ref[g, 0:NUM, :] = num_ref[g]
        for c in range(N_CAT):
            h, s = c % 4, c // 4
            out_ref[g, NUM + EDIM * c:NUM + EDIM * (c + 1), :] = (
                vt[g, EDIM * h:EDIM * (h + 1), IP * s:IP * s + T])


def kernel(numerical, categorical, tables):
    info = plsc.get_sparse_core_info()
    nw = info.num_cores * info.num_subcores
    tab_flat = tables.reshape(N_CAT * VOCAB, EDIM)
    cat_prep = pl.pallas_call(
        _idx_body,
        grid=(B // _GI,),
        in_specs=[pl.BlockSpec((_GI, N_CAT, T), lambda i: (i, 0, 0))],
        out_specs=pl.BlockSpec((_GI * NSTR, SROWS), lambda i: (i, 0)),
        out_shape=jax.ShapeDtypeStruct((B * NSTR, SROWS), jnp.int32),
    )(categorical)
    x = _make_sc_gather(nw)(cat_prep, tab_flat)
    out = pl.pallas_call(
        _tc_body,
        grid=(B // _GB,),
        in_specs=[
            pl.BlockSpec((_GB * XD, 128), lambda i: (i, 0)),
            pl.BlockSpec((_GB, NUM, T), lambda i: (i, 0, 0)),
        ],
        out_specs=pl.BlockSpec((_GB, OUT_F, T), lambda i: (i, 0, 0)),
        out_shape=jax.ShapeDtypeStruct((B, OUT_F, T), jnp.float32),
    )(x.reshape(B * XD, 128), numerical)
    return out


# speedup vs baseline: 1.5332x; 1.0109x over previous
import functools

import jax
import jax.numpy as jnp
from jax import lax
from jax.experimental import pallas as pl
from jax.experimental.pallas import tpu as pltpu
from jax.experimental.pallas import tpu_sc as plsc

B, T = 1024, 50
NUM = 16
N_CAT = 26
VOCAB = 100000
EDIM = 32
OUT_F = NUM + N_CAT * EDIM
IP = 64
NCV = 28
NSLOT = NCV // 4
IB = NCV * IP
SROWS = 128
NSTR = IB // SROWS
XD = IB * EDIM // 128

_GI = 32


def _idx_body(cat_ref, out_ref):
    y = cat_ref[...] + lax.broadcasted_iota(jnp.int32, (_GI, N_CAT, T), 1) * VOCAB
    padt = lax.broadcasted_iota(jnp.int32, (_GI, IP - T), 1) + T
    dummy = lax.broadcasted_iota(jnp.int32, (_GI, IP), 1)
    pieces = []
    for s in range(NSLOT):
        for h in range(4):
            c = h + 4 * s
            if c < N_CAT:
                pieces.append(y[:, c, :])
                pieces.append(padt + c * VOCAB)
            else:
                pieces.append(dummy + (c - N_CAT) * IP)
    nat = jnp.concatenate(pieces, axis=1).astype(jnp.float32)
    a = lax.broadcasted_iota(jnp.int32, (4 * IP, 4 * IP), 0)
    w = lax.broadcasted_iota(jnp.int32, (4 * IP, 4 * IP), 1)
    perm = jnp.where(w == 4 * (a % IP) + a // IP, 1.0, 0.0)
    q = lax.dot_general(
        nat.reshape(_GI * NSLOT, 4 * IP), perm, (((1,), (0,)), ((), ())),
        precision=lax.Precision.HIGHEST,
        preferred_element_type=jnp.float32,
    )
    out_ref[...] = q.astype(jnp.int32).reshape(_GI * NSTR, SROWS)


def _make_sc_gather(num_workers: int):
    b_per_w = B // num_workers
    mesh = plsc.VectorSubcoreMesh(
        core_axis_name="c", subcore_axis_name="s", num_cores=2)

    @functools.partial(
        pl.kernel,
        mesh=mesh,
        compiler_params=pltpu.CompilerParams(use_tc_tiling_on_sc=False),
        out_type=jax.ShapeDtypeStruct((B * IB, EDIM), jnp.float32),
        scratch_types=[
            pltpu.VMEM((NSTR, SROWS), jnp.int32),
            pltpu.VMEM((2, IB, EDIM), jnp.float32),
            pltpu.SemaphoreType.DMA,
            pltpu.SemaphoreType.DMA,
        ],
    )
    def k(cat_hbm, tab_hbm, x_hbm, idx_v, vbuf, gsem, wsem):
        nc = plsc.get_sparse_core_info().num_cores
        wid = lax.axis_index("s") * nc + lax.axis_index("c")
        b0 = wid * b_per_w

        def load_and_fire(b, p):
            pltpu.sync_copy(cat_hbm.at[pl.ds(b * NSTR, NSTR)], idx_v)

            def fire_s(s, cc):
                pltpu.async_copy(
                    tab_hbm.at[idx_v.at[s]],
                    vbuf.at[p, pl.ds(s * SROWS, SROWS)],
                    gsem,
                )
                return cc

            lax.fori_loop(0, NSTR, fire_s, 0)

        load_and_fire(b0, 0)

        def body_b(bi, carry):
            b = b0 + bi
            p = lax.rem(bi, 2)
            q = 1 - p
            xslc = x_hbm.at[pl.ds(b * IB, IB)]
            pltpu.make_async_copy(xslc, vbuf.at[p], gsem).wait()
            pltpu.async_copy(vbuf.at[p], xslc, wsem)

            @pl.when(bi < b_per_w - 1)
            def _():
                @pl.when(bi > 0)
                def _():
                    pltpu.make_async_copy(vbuf.at[q], xslc, wsem).wait()

                load_and_fire(b + 1, q)

            return carry

        lax.fori_loop(0, b_per_w, body_b, 0)
        xsl0 = x_hbm.at[pl.ds(b0 * IB, IB)]
        pltpu.make_async_copy(vbuf.at[0], xsl0, wsem).wait()
        pltpu.make_async_copy(vbuf.at[1], xsl0, wsem).wait()

    return k


_GB = 16


def _tc_body(x_ref, num_ref, out_ref):
    v = x_ref[...].reshape(_GB, XD, 128)
    vt = jnp.swapaxes(v, 1, 2)
    for g in range(_GB):
        out_ref[g, 0:NUM, :] = num_ref[g]
        for c in range(N_CAT):
            h, s = c % 4, c // 4
            out_ref[g, NUM + EDIM * c:NUM + EDIM * (c + 1), :] = (
                vt[g, EDIM * h:EDIM * (h + 1), IP * s:IP * s + T])


def kernel(numerical, categorical, tables):
    info = plsc.get_sparse_core_info()
    nw = info.num_cores * info.num_subcores
    tab_flat = tables.reshape(N_CAT * VOCAB, EDIM)
    cat_prep = pl.pallas_call(
        _idx_body,
        grid=(B // _GI,),
        in_specs=[pl.BlockSpec((_GI, N_CAT, T), lambda i: (i, 0, 0))],
        out_specs=pl.BlockSpec((_GI * NSTR, SROWS), lambda i: (i, 0)),
        out_shape=jax.ShapeDtypeStruct((B * NSTR, SROWS), jnp.int32),
    )(categorical)
    x = _make_sc_gather(nw)(cat_prep, tab_flat)
    out = pl.pallas_call(
        _tc_body,
        grid=(B // _GB,),
        in_specs=[
            pl.BlockSpec((_GB * XD, 128), lambda i: (i, 0)),
            pl.BlockSpec((_GB, NUM, T), lambda i: (i, 0, 0)),
        ],
        out_specs=pl.BlockSpec((_GB, OUT_F, T), lambda i: (i, 0, 0)),
        out_shape=jax.ShapeDtypeStruct((B, OUT_F, T), jnp.float32),
    )(x.reshape(B * XD, 128), numerical)
    return out


# speedup vs baseline: 1.5351x; 1.0013x over previous
import functools

import jax
import jax.numpy as jnp
from jax import lax
from jax.experimental import pallas as pl
from jax.experimental.pallas import tpu as pltpu
from jax.experimental.pallas import tpu_sc as plsc

B, T = 1024, 50
NUM = 16
N_CAT = 26
VOCAB = 100000
EDIM = 32
OUT_F = NUM + N_CAT * EDIM
IP = 64
NCV = 28
NSLOT = NCV // 4
IB = NCV * IP
SROWS = 128
NSTR = IB // SROWS
XD = IB * EDIM // 128

_GI = 32


def _idx_body(cat_ref, out_ref):
    y = cat_ref[...] + lax.broadcasted_iota(jnp.int32, (_GI, N_CAT, T), 1) * VOCAB
    padt = lax.broadcasted_iota(jnp.int32, (_GI, IP - T), 1) + T
    dummy = lax.broadcasted_iota(jnp.int32, (_GI, IP), 1)
    pieces = []
    for s in range(NSLOT):
        for h in range(4):
            c = h + 4 * s
            if c < N_CAT:
                pieces.append(y[:, c, :])
                pieces.append(padt + c * VOCAB)
            else:
                pieces.append(dummy + (c - N_CAT) * IP)
    nat = jnp.concatenate(pieces, axis=1).astype(jnp.float32)
    a = lax.broadcasted_iota(jnp.int32, (4 * IP, 4 * IP), 0)
    w = lax.broadcasted_iota(jnp.int32, (4 * IP, 4 * IP), 1)
    perm = jnp.where(w == 4 * (a % IP) + a // IP, 1.0, 0.0)
    q = lax.dot_general(
        nat.reshape(_GI * NSLOT, 4 * IP), perm, (((1,), (0,)), ((), ())),
        precision=lax.Precision.HIGHEST,
        preferred_element_type=jnp.float32,
    )
    out_ref[...] = q.astype(jnp.int32).reshape(_GI * NSTR, SROWS)


def _make_sc_gather(num_workers: int):
    b_per_w = B // num_workers
    mesh = plsc.VectorSubcoreMesh(
        core_axis_name="c", subcore_axis_name="s", num_cores=2)

    @functools.partial(
        pl.kernel,
        mesh=mesh,
        compiler_params=pltpu.CompilerParams(use_tc_tiling_on_sc=False),
        out_type=jax.ShapeDtypeStruct((B * IB, EDIM), jnp.float32),
        scratch_types=[
            pltpu.VMEM((NSTR, SROWS), jnp.int32),
            pltpu.VMEM((2, IB, EDIM), jnp.float32),
            pltpu.SemaphoreType.DMA,
            pltpu.SemaphoreType.DMA,
        ],
    )
    def k(cat_hbm, tab_hbm, x_hbm, idx_v, vbuf, gsem, wsem):
        nc = plsc.get_sparse_core_info().num_cores
        wid = lax.axis_index("s") * nc + lax.axis_index("c")
        b0 = wid * b_per_w

        def load_and_fire(b, p):
            pltpu.sync_copy(cat_hbm.at[pl.ds(b * NSTR, NSTR)], idx_v)

            def fire_s(s, cc):
                pltpu.async_copy(
                    tab_hbm.at[idx_v.at[s]],
                    vbuf.at[p, pl.ds(s * SROWS, SROWS)],
                    gsem,
                )
                return cc

            lax.fori_loop(0, NSTR, fire_s, 0)

        load_and_fire(b0, 0)

        def body_b(bi, carry):
            b = b0 + bi
            p = lax.rem(bi, 2)
            q = 1 - p
            xslc = x_hbm.at[pl.ds(b * IB, IB)]
            pltpu.make_async_copy(xslc, vbuf.at[p], gsem).wait()
            pltpu.async_copy(vbuf.at[p], xslc, wsem)

            @pl.when(bi < b_per_w - 1)
            def _():
                @pl.when(bi > 0)
                def _():
                    pltpu.make_async_copy(vbuf.at[q], xslc, wsem).wait()

                load_and_fire(b + 1, q)

            return carry

        lax.fori_loop(0, b_per_w, body_b, 0)
        xsl0 = x_hbm.at[pl.ds(b0 * IB, IB)]
        pltpu.make_async_copy(vbuf.at[0], xsl0, wsem).wait()
        pltpu.make_async_copy(vbuf.at[1], xsl0, wsem).wait()

    return k


_GB = 32


def _tc_body(x_ref, num_ref, out_ref):
    v = x_ref[...].reshape(_GB, XD, 128)
    vt = jnp.swapaxes(v, 1, 2)
    for g in range(_GB):
        out_ref[g, 0:NUM, :] = num_ref[g]
        for c in range(N_CAT):
            h, s = c % 4, c // 4
            out_ref[g, NUM + EDIM * c:NUM + EDIM * (c + 1), :] = (
                vt[g, EDIM * h:EDIM * (h + 1), IP * s:IP * s + T])


def kernel(numerical, categorical, tables):
    info = plsc.get_sparse_core_info()
    nw = info.num_cores * info.num_subcores
    tab_flat = tables.reshape(N_CAT * VOCAB, EDIM)
    cat_prep = pl.pallas_call(
        _idx_body,
        grid=(B // _GI,),
        in_specs=[pl.BlockSpec((_GI, N_CAT, T), lambda i: (i, 0, 0))],
        out_specs=pl.BlockSpec((_GI * NSTR, SROWS), lambda i: (i, 0)),
        out_shape=jax.ShapeDtypeStruct((B * NSTR, SROWS), jnp.int32),
    )(categorical)
    x = _make_sc_gather(nw)(cat_prep, tab_flat)
    out = pl.pallas_call(
        _tc_body,
        grid=(B // _GB,),
        in_specs=[
            pl.BlockSpec((_GB * XD, 128), lambda i: (i, 0)),
            pl.BlockSpec((_GB, NUM, T), lambda i: (i, 0, 0)),
        ],
        out_specs=pl.BlockSpec((_GB, OUT_F, T), lambda i: (i, 0, 0)),
        out_shape=jax.ShapeDtypeStruct((B, OUT_F, T), jnp.float32),
    )(x.reshape(B * XD, 128), numerical)
    return out
